# Initial kernel scaffold; baseline (speedup 1.0000x reference)
import jax
import jax.numpy as jnp
from jax.experimental import pallas as pl


def kernel(input_ids, attention_mask, embedding_dict, input_ids2dict_ids):
    raise NotImplementedError("write your pallas kernel here")


# speedup vs baseline: 6.9015x; 6.9015x over previous
import functools

import jax
import jax.numpy as jnp
from jax import lax
from jax.experimental import pallas as pl
from jax.experimental.pallas import tpu as pltpu
from jax.experimental.pallas import tpu_sc as plsc

DICT_SIZE = 100000
DIM = 128

NC = 2
NS = 16
NW = NC * NS

K = 128


def _body(n_tok, b_per_w, n_chunks,
          ids_hbm, remap_hbm, emb_hbm, out_hbm,
          ids_v, dict_v, rows_v, sem, sem2):
    wid = lax.axis_index("s") * NC + lax.axis_index("c")
    base = wid * b_per_w

    pltpu.sync_copy(ids_hbm.at[pl.ds(base, b_per_w)], ids_v)

    def remap_step(j, carry):
        off = pl.multiple_of(j * K, K)
        pltpu.async_copy(remap_hbm.at[ids_v.at[pl.ds(off, K)]],
                         dict_v.at[pl.ds(off, K)], sem).wait()
        return carry

    lax.fori_loop(0, n_chunks, remap_step, 0)

    def row_step(j, carry):
        off = pl.multiple_of(j * K, K)
        pltpu.async_copy(emb_hbm.at[dict_v.at[pl.ds(off, K)]],
                         rows_v, sem2).wait()
        pltpu.sync_copy(rows_v, out_hbm.at[pl.ds(base + off, K)])
        return carry

    lax.fori_loop(0, n_chunks, row_step, 0)


@functools.partial(jax.jit, static_argnums=())
def _lookup(ids_flat, remap, emb):
    n_tok = ids_flat.shape[0]
    b_per_w = n_tok // NW
    n_chunks = b_per_w // K
    mesh = plsc.VectorSubcoreMesh(core_axis_name="c", subcore_axis_name="s")
    fn = pl.kernel(
        functools.partial(_body, n_tok, b_per_w, n_chunks),
        out_type=jax.ShapeDtypeStruct((n_tok, DIM), jnp.float32),
        mesh=mesh,
        scratch_types=[
            pltpu.VMEM((b_per_w,), jnp.int32),
            pltpu.VMEM((b_per_w,), jnp.int32),
            pltpu.VMEM((K, DIM), jnp.float32),
            pltpu.SemaphoreType.DMA,
            pltpu.SemaphoreType.DMA,
        ],
    )
    return fn(ids_flat, remap, emb)


def kernel(input_ids, attention_mask, embedding_dict, input_ids2dict_ids):
    batch, seq = input_ids.shape
    ids_flat = input_ids.reshape(-1).astype(jnp.int32)
    remap = input_ids2dict_ids.astype(jnp.int32)
    out = _lookup(ids_flat, remap, embedding_dict)
    return (out.reshape(batch, seq, DIM), attention_mask)


# speedup vs baseline: 8.4234x; 1.2205x over previous
import functools

import jax
import jax.numpy as jnp
from jax import lax
from jax.experimental import pallas as pl
from jax.experimental.pallas import tpu as pltpu
from jax.experimental.pallas import tpu_sc as plsc

DIM = 128

NC = 2
NS = 16
NW = NC * NS

K = 128
NBUF = 5


def _body(b_per_w, n_chunks,
          ids_hbm, remap_hbm, emb_hbm, out_hbm,
          ids_v, dict_v, r0, r1, r2, r3, r4,
          sem_r, sg0, sg1, sg2, sg3, sg4, ss0, ss1, ss2, ss3, ss4):
    rows = (r0, r1, r2, r3, r4)
    sg = (sg0, sg1, sg2, sg3, sg4)
    ss = (ss0, ss1, ss2, ss3, ss4)

    wid = lax.axis_index("s") * NC + lax.axis_index("c")
    base = wid * b_per_w

    pltpu.sync_copy(ids_hbm.at[pl.ds(base, b_per_w)], ids_v)

    def fire_remap(j, c):
        off = pl.multiple_of(j * K, K)
        pltpu.async_copy(remap_hbm.at[ids_v.at[pl.ds(off, K)]],
                         dict_v.at[pl.ds(off, K)], sem_r)
        return c

    lax.fori_loop(0, n_chunks, fire_remap, 0)

    def drain_remap(j, c):
        off = pl.multiple_of(j * K, K)
        pltpu.make_async_copy(remap_hbm.at[ids_v.at[pl.ds(off, K)]],
                              dict_v.at[pl.ds(off, K)], sem_r).wait()
        return c

    lax.fori_loop(0, n_chunks, drain_remap, 0)

    def issue_g(j, slot):
        off = pl.multiple_of(j * K, K)
        pltpu.async_copy(emb_hbm.at[dict_v.at[pl.ds(off, K)]],
                         rows[slot], sg[slot])

    def wait_g(slot):
        pltpu.make_async_copy(emb_hbm.at[dict_v.at[pl.ds(0, K)]],
                              rows[slot], sg[slot]).wait()

    def issue_s(j, slot):
        off = pl.multiple_of(j * K, K)
        pltpu.async_copy(rows[slot], out_hbm.at[pl.ds(base + off, K)],
                         ss[slot])

    def wait_s(slot):
        pltpu.make_async_copy(rows[slot], out_hbm.at[pl.ds(base, K)],
                              ss[slot]).wait()

    for m in range(NBUF - 1):
        issue_g(m, m)
    wait_g(0)
    issue_s(0, 0)
    issue_g(NBUF - 1, NBUF - 1)

    def group(g, c):
        for b in range(NBUF):
            j = g * NBUF + 1 + b
            slot = (1 + b) % NBUF
            prev = b % NBUF
            wait_g(slot)
            issue_s(j, slot)
            wait_s(prev)
            issue_g(j + NBUF - 1, prev)
        return c

    lax.fori_loop(0, (n_chunks - NBUF) // NBUF, group, 0)

    for j in range(n_chunks - NBUF + 1, n_chunks):
        slot = j % NBUF
        wait_g(slot)
        issue_s(j, slot)
        wait_s((j - 1) % NBUF)
    wait_s((n_chunks - 1) % NBUF)


@jax.jit
def _lookup(ids_flat, remap, emb):
    n_tok = ids_flat.shape[0]
    b_per_w = n_tok // NW
    n_chunks = b_per_w // K
    assert n_tok == b_per_w * NW and b_per_w == n_chunks * K
    assert (n_chunks - NBUF) % NBUF == 0
    mesh = plsc.VectorSubcoreMesh(core_axis_name="c", subcore_axis_name="s")
    fn = pl.kernel(
        functools.partial(_body, b_per_w, n_chunks),
        out_type=jax.ShapeDtypeStruct((n_tok, DIM), jnp.float32),
        mesh=mesh,
        scratch_types=(
            [pltpu.VMEM((b_per_w,), jnp.int32),
             pltpu.VMEM((b_per_w,), jnp.int32)]
            + [pltpu.VMEM((K, DIM), jnp.float32)] * NBUF
            + [pltpu.SemaphoreType.DMA] * (1 + 2 * NBUF)
        ),
    )
    return fn(ids_flat, remap, emb)


def kernel(input_ids, attention_mask, embedding_dict, input_ids2dict_ids):
    batch, seq = input_ids.shape
    ids_flat = input_ids.reshape(-1).astype(jnp.int32)
    remap = input_ids2dict_ids.astype(jnp.int32)
    out = _lookup(ids_flat, remap, embedding_dict)
    return (out.reshape(batch, seq, DIM), attention_mask)
